# baseline (speedup 1.0000x reference)
import functools

import jax
import jax.numpy as jnp
from jax import lax
from jax.experimental import pallas as pl
from jax.experimental.pallas import tpu as pltpu
from jax.experimental.pallas import tpu_sc as plsc
from jax.experimental import layout as jlayout

NC, NS, L = 2, 16, 16
NW = NC * NS

D = 64
WPR = D // 2

V = 1_000_000
B = 4096
H = 200

RB = 8192
RB4 = RB // 4
NBLK = (V + RB - 1) // RB
VS = NBLK * RB


def _rne_bf16_bits(x):
    xi = jax.lax.bitcast_convert_type(x, jnp.uint32)
    return (xi + jnp.uint32(0x7FFF) + ((xi >> 16) & jnp.uint32(1))) >> 16


def _pack_tc_body(in_ref, out_ref):
    xlo = in_ref[0:WPR, :]
    xhi = in_ref[WPR:D, :]
    w = _rne_bf16_bits(xlo) | (_rne_bf16_bits(xhi) << 16)
    wq = jnp.concatenate(
        [w[:, p * RB4 : (p + 1) * RB4] for p in range(4)], axis=0
    )
    out_ref[...] = jax.lax.bitcast_convert_type(jnp.transpose(wq), jnp.int32)


def _pack_table(tabT):
    return pl.pallas_call(
        _pack_tc_body,
        grid=(NBLK,),
        in_specs=[pl.BlockSpec((D, RB), lambda i: (0, i))],
        out_specs=pl.BlockSpec((RB4, 128), lambda i: (i, 0)),
        out_shape=jax.ShapeDtypeStruct((NBLK * RB4, 128), jnp.int32),
    )(tabT)


GB = 128
NBUF = 8
BLOCKS = H * (B // GB)
BPW = BLOCKS // NW
GROUPS = BPW // NBUF
IPW = BPW * GB


def _slot_map_all(idx_v):
    sh = RB4.bit_length() - 1

    def body(g, _):
        v = idx_v[pl.ds(g * L, L)]
        hi = v & jnp.int32(-RB)
        q4 = (v & jnp.int32(RB4 - 1)) << 2
        p = (v >> sh) & jnp.int32(3)
        idx_v[pl.ds(g * L, L)] = hi | q4 | p
        return ()

    lax.fori_loop(0, IPW // L, body, (), unroll=4)


def _cpair_block(rows_v, tr_v, col0):
    iota = lax.iota(jnp.int32, L)
    iota2 = iota * 2
    m16 = jnp.int32(0xFFFF)

    def row_body(r, _):
        csp = jnp.full((L,), col0 + r, dtype=jnp.int32)
        rsp = jnp.full((L,), r, dtype=jnp.int32)
        we = plsc.load_gather(rows_v, [rsp, iota2])
        wo = plsc.load_gather(rows_v, [rsp, iota2 + 1])
        lo = (we & m16) | (wo << 16)
        hi = lax.shift_right_logical(we, 16) | (wo & ~m16)
        plsc.store_scatter(tr_v, [iota2, csp], lo)
        plsc.store_scatter(tr_v, [iota2 + 1, csp], lo)
        plsc.store_scatter(tr_v, [iota2 + 2 * L, csp], hi)
        plsc.store_scatter(tr_v, [iota2 + (2 * L + 1), csp], hi)
        return ()

    lax.fori_loop(0, GB, row_body, (), unroll=2)


QB = 4
NQ = BPW // QB


def _gather_body(idx_hbm, wtab_hbm, out_hbm, idx_v, rows_v, tr_v, sems):
    isem, gsems, osems = sems
    wid = lax.axis_index("s") * NC + lax.axis_index("c")
    base = wid * BPW

    def gather_copy(t, slot):
        return pltpu.make_async_copy(
            wtab_hbm.at[idx_v.at[pl.ds((t - base) * GB, GB)]],
            rows_v.at[slot],
            gsems[slot],
        )

    def out_copy(q, qslot):
        t0 = base + q * QB
        h = t0 // (B // GB)
        j0 = t0 % (B // GB)
        return pltpu.make_async_copy(
            tr_v.at[qslot],
            out_hbm.at[h, :, pl.ds(j0 * GB, QB * GB)],
            osems[qslot],
        )

    pltpu.make_async_copy(
        idx_hbm.at[pl.ds(wid * IPW, IPW)], idx_v, isem
    ).start()
    pltpu.make_async_copy(
        idx_hbm.at[pl.ds(wid * IPW, IPW)], idx_v, isem
    ).wait()
    _slot_map_all(idx_v)

    for s in range(NBUF):
        gather_copy(base + s, s).start()

    def pair_body(g, _):
        for q2 in range(2):
            q = g * 2 + q2

            @pl.when(q >= 2)
            def _():
                out_copy(q - 2, q2).wait()

            for jj in range(QB):
                t = base + q * QB + jj
                slot = q2 * QB + jj
                gather_copy(t, slot).wait()
                _cpair_block(rows_v.at[slot], tr_v.at[q2], jj * GB)

                @pl.when(q * QB + jj < BPW - NBUF)
                def _():
                    gather_copy(t + NBUF, slot).start()

            out_copy(q, q2).start()
        return ()

    lax.fori_loop(0, NQ // 2, pair_body, (), unroll=False)
    out_copy(NQ - 2, 0).wait()
    out_copy(NQ - 1, 1).wait()


def _gather(idx_flat, wtab2):
    mesh = plsc.VectorSubcoreMesh(
        core_axis_name="c", subcore_axis_name="s", num_cores=NC, num_subcores=NS
    )
    kfn = pl.kernel(
        _gather_body,
        out_type=jax.ShapeDtypeStruct((H, D, B), jnp.int32),
        mesh=mesh,
        scratch_types=[
            pltpu.VMEM((IPW,), jnp.int32),
            pltpu.VMEM((NBUF, GB, WPR), jnp.int32),
            pltpu.VMEM((2, D, QB * GB), jnp.int32),
            (
                pltpu.SemaphoreType.DMA,
                [pltpu.SemaphoreType.DMA] * NBUF,
                [pltpu.SemaphoreType.DMA] * 2,
            ),
        ],
        compiler_params=pltpu.CompilerParams(
            needs_layout_passes=False, use_tc_tiling_on_sc=False
        ),
    )
    return kfn(idx_flat, wtab2)


@jax.jit
def _run(input, embedding_weight):
    tabT = jnp.transpose(embedding_weight)
    wtab = _pack_table(tabT)
    wtab2 = wtab.reshape(VS, WPR)
    idx_flat = jnp.transpose(input).reshape(B * H)
    outw = _gather(idx_flat, wtab2)
    ou = jax.lax.bitcast_convert_type(outw, jnp.uint32)
    sh = ((jnp.arange(D, dtype=jnp.uint32) & 1) * 16).reshape(1, D, 1)
    v16 = (ou >> sh).astype(jnp.uint16)
    bf = jax.lax.bitcast_convert_type(v16, jnp.bfloat16)
    return jnp.transpose(bf, (2, 0, 1))


def kernel(input, embedding_weight):
    return _run(input, embedding_weight)


# speedup vs baseline: 1.5260x; 1.5260x over previous
import functools

import jax
import jax.numpy as jnp
from jax import lax
from jax.experimental import pallas as pl
from jax.experimental.pallas import tpu as pltpu
from jax.experimental.pallas import tpu_sc as plsc
from jax.experimental import layout as jlayout

NC, NS, L = 2, 16, 16
NW = NC * NS

D = 64
WPR = D // 2

V = 1_000_000
B = 4096
H = 200

RB = 8192
RB4 = RB // 4
NBLK = (V + RB - 1) // RB
VS = NBLK * RB


def _rne_bf16_bits(x):
    xi = jax.lax.bitcast_convert_type(x, jnp.uint32)
    return (xi + jnp.uint32(0x7FFF) + ((xi >> 16) & jnp.uint32(1))) >> 16


def _pack_tc_body(in_ref, out_ref):
    xlo = in_ref[0:WPR, :]
    xhi = in_ref[WPR:D, :]
    w = _rne_bf16_bits(xlo) | (_rne_bf16_bits(xhi) << 16)
    wq = jnp.concatenate(
        [w[:, p * RB4 : (p + 1) * RB4] for p in range(4)], axis=0
    )
    out_ref[...] = jax.lax.bitcast_convert_type(jnp.transpose(wq), jnp.int32)


def _pack_table(tabT):
    return pl.pallas_call(
        _pack_tc_body,
        grid=(NBLK,),
        in_specs=[pl.BlockSpec((D, RB), lambda i: (0, i))],
        out_specs=pl.BlockSpec((RB4, 128), lambda i: (i, 0)),
        out_shape=jax.ShapeDtypeStruct((NBLK * RB4, 128), jnp.int32),
    )(tabT)


GB = 128
NBUF = 8
BLOCKS = H * (B // GB)
BPW = BLOCKS // NW
GROUPS = BPW // NBUF
IPW = BPW * GB


def _slot_map_all(idx_v):
    sh = RB4.bit_length() - 1

    def body(g, _):
        v = idx_v[pl.ds(g * L, L)]
        hi = v & jnp.int32(-RB)
        q4 = (v & jnp.int32(RB4 - 1)) << 2
        p = (v >> sh) & jnp.int32(3)
        idx_v[pl.ds(g * L, L)] = hi | q4 | p
        return ()

    lax.fori_loop(0, IPW // L, body, (), unroll=4)


def _cpair_block(rows_v, tr_v):
    iota2 = lax.iota(jnp.int32, L) * 2
    m16 = jnp.int32(0xFFFF)

    def row_body(r, _):
        rsp = jnp.full((L,), r, dtype=jnp.int32)
        we = plsc.load_gather(rows_v, [rsp, iota2])
        wo = plsc.load_gather(rows_v, [rsp, iota2 + 1])
        lo = (we & m16) | (wo << 16)
        hi = lax.shift_right_logical(we, 16) | (wo & ~m16)
        tr_v[r, pl.ds(0, 2 * L)] = plsc.bitcast(lo, jnp.bfloat16)
        tr_v[r, pl.ds(2 * L, 2 * L)] = plsc.bitcast(hi, jnp.bfloat16)
        return ()

    lax.fori_loop(0, GB, row_body, (), unroll=2)


def _gather_body(idx_hbm, wtab_hbm, out_hbm, idx_v, rows_v, tr_v, sems):
    isem, gsems, osems = sems
    wid = lax.axis_index("s") * NC + lax.axis_index("c")
    base = wid * BPW

    def gather_copy(t, slot):
        return pltpu.make_async_copy(
            wtab_hbm.at[idx_v.at[pl.ds((t - base) * GB, GB)]],
            rows_v.at[slot],
            gsems[slot],
        )

    def out_copy(t, slot):
        h = t // (B // GB)
        j = t % (B // GB)
        return pltpu.make_async_copy(
            tr_v.at[slot],
            out_hbm.at[pl.ds(j * GB, GB), h, :],
            osems[slot],
        )

    pltpu.make_async_copy(
        idx_hbm.at[pl.ds(wid * IPW, IPW)], idx_v, isem
    ).start()
    pltpu.make_async_copy(
        idx_hbm.at[pl.ds(wid * IPW, IPW)], idx_v, isem
    ).wait()
    _slot_map_all(idx_v)

    for s in range(NBUF):
        gather_copy(base + s, s).start()

    def group_body(g, _):
        t0 = base + g * NBUF
        for s in range(NBUF):
            t = t0 + s
            gather_copy(t, s).wait()

            @pl.when(g > 0)
            def _():
                out_copy(t - NBUF, s).wait()

            _cpair_block(rows_v.at[s], tr_v.at[s])
            out_copy(t, s).start()

            @pl.when(g < GROUPS - 1)
            def _():
                gather_copy(t + NBUF, s).start()

        return ()

    lax.fori_loop(0, GROUPS, group_body, (), unroll=False)
    for s in range(NBUF):
        out_copy(base + (GROUPS - 1) * NBUF + s, s).wait()


def _gather(idx_flat, wtab2):
    mesh = plsc.VectorSubcoreMesh(
        core_axis_name="c", subcore_axis_name="s", num_cores=NC, num_subcores=NS
    )
    kfn = pl.kernel(
        _gather_body,
        out_type=jax.ShapeDtypeStruct((B, H, D), jnp.bfloat16),
        mesh=mesh,
        scratch_types=[
            pltpu.VMEM((IPW,), jnp.int32),
            pltpu.VMEM((NBUF, GB, WPR), jnp.int32),
            pltpu.VMEM((NBUF, GB, D), jnp.bfloat16),
            (
                pltpu.SemaphoreType.DMA,
                [pltpu.SemaphoreType.DMA] * NBUF,
                [pltpu.SemaphoreType.DMA] * NBUF,
            ),
        ],
        compiler_params=pltpu.CompilerParams(
            needs_layout_passes=False, use_tc_tiling_on_sc=False
        ),
    )
    return kfn(idx_flat, wtab2)


@jax.jit
def _run(input, embedding_weight):
    tabT = jnp.transpose(embedding_weight)
    wtab = _pack_table(tabT)
    wtab2 = wtab.reshape(VS, WPR)
    idx_flat = jnp.transpose(input).reshape(B * H)
    return _gather(idx_flat, wtab2)


def kernel(input, embedding_weight):
    return _run(input, embedding_weight)


# speedup vs baseline: 1.5270x; 1.0007x over previous
import jax
import jax.numpy as jnp
from jax import lax
from jax.experimental import pallas as pl
from jax.experimental.pallas import tpu as pltpu
from jax.experimental.pallas import tpu_sc as plsc

NC, NS, L = 2, 16, 16
NW = NC * NS

D = 64
WPR = D // 2

V = 1_000_000
B = 4096
H = 200

RB = 8192
RB4 = RB // 4
NBLK = (V + RB - 1) // RB
VS = NBLK * RB


def _rne_bf16_bits(x):
    xi = jax.lax.bitcast_convert_type(x, jnp.uint32)
    return (xi + jnp.uint32(0x7FFF) + ((xi >> 16) & jnp.uint32(1))) >> 16


def _pack_tc_body(in_ref, out_ref):
    xlo = in_ref[0:WPR, :]
    xhi = in_ref[WPR:D, :]
    w = _rne_bf16_bits(xlo) | (_rne_bf16_bits(xhi) << 16)
    wq = jnp.concatenate(
        [w[:, p * RB4 : (p + 1) * RB4] for p in range(4)], axis=0
    )
    out_ref[...] = jax.lax.bitcast_convert_type(jnp.transpose(wq), jnp.int32)


def _pack_table(tabT):
    return pl.pallas_call(
        _pack_tc_body,
        grid=(NBLK,),
        in_specs=[pl.BlockSpec((D, RB), lambda i: (0, i))],
        out_specs=pl.BlockSpec((RB4, 128), lambda i: (i, 0)),
        out_shape=jax.ShapeDtypeStruct((NBLK * RB4, 128), jnp.int32),
    )(tabT)


GB = 128
NBUF = 8
BLOCKS = H * (B // GB)
BPW = BLOCKS // NW
GROUPS = BPW // NBUF
IPW = BPW * GB


def _slot_map_all(idx_v):
    sh = RB4.bit_length() - 1

    def body(g, _):
        v = idx_v[pl.ds(g * L, L)]
        hi = v & jnp.int32(-RB)
        q4 = (v & jnp.int32(RB4 - 1)) << 2
        p = (v >> sh) & jnp.int32(3)
        idx_v[pl.ds(g * L, L)] = hi | q4 | p
        return ()

    lax.fori_loop(0, IPW // L, body, (), unroll=4)


def _cpair_block(rows_v, tr_v):
    iota2 = lax.iota(jnp.int32, L) * 2
    m16 = jnp.int32(0xFFFF)

    def row_body(r, _):
        rsp = jnp.full((L,), r, dtype=jnp.int32)
        we = plsc.load_gather(rows_v, [rsp, iota2])
        wo = plsc.load_gather(rows_v, [rsp, iota2 + 1])
        lo = (we & m16) | (wo << 16)
        hi = lax.shift_right_logical(we, 16) | (wo & ~m16)
        tr_v[r, pl.ds(0, 2 * L)] = plsc.bitcast(lo, jnp.bfloat16)
        tr_v[r, pl.ds(2 * L, 2 * L)] = plsc.bitcast(hi, jnp.bfloat16)
        return ()

    lax.fori_loop(0, GB, row_body, (), unroll=2)


def _gather_body(idx_hbm, wtab_hbm, out_hbm, idx_v, rows_v, tr_v, sems):
    isem, gsems, osems = sems
    wid = lax.axis_index("s") * NC + lax.axis_index("c")
    base = wid * BPW

    def gather_copy(t, slot):
        return pltpu.make_async_copy(
            wtab_hbm.at[idx_v.at[pl.ds((t - base) * GB, GB)]],
            rows_v.at[slot],
            gsems[slot],
        )

    def out_copy(t, slot):
        h = t // (B // GB)
        j = t % (B // GB)
        return pltpu.make_async_copy(
            tr_v.at[slot],
            out_hbm.at[pl.ds(j * GB, GB), h, :],
            osems[slot],
        )

    pltpu.make_async_copy(
        idx_hbm.at[pl.ds(wid * IPW, IPW)], idx_v, isem
    ).start()
    pltpu.make_async_copy(
        idx_hbm.at[pl.ds(wid * IPW, IPW)], idx_v, isem
    ).wait()
    _slot_map_all(idx_v)

    for s in range(NBUF):
        gather_copy(base + s, s).start()

    def group_body(g, _):
        t0 = base + g * NBUF
        for s in range(NBUF):
            t = t0 + s
            gather_copy(t, s).wait()

            @pl.when(g > 0)
            def _():
                out_copy(t - NBUF, s).wait()

            _cpair_block(rows_v.at[s], tr_v.at[s])
            out_copy(t, s).start()

            @pl.when(g < GROUPS - 1)
            def _():
                gather_copy(t + NBUF, s).start()

        return ()

    lax.fori_loop(0, GROUPS, group_body, (), unroll=False)
    for s in range(NBUF):
        out_copy(base + (GROUPS - 1) * NBUF + s, s).wait()


def _gather(idx_flat, wtab2):
    mesh = plsc.VectorSubcoreMesh(
        core_axis_name="c", subcore_axis_name="s", num_cores=NC, num_subcores=NS
    )
    kfn = pl.kernel(
        _gather_body,
        out_type=jax.ShapeDtypeStruct((B, H, D), jnp.bfloat16),
        mesh=mesh,
        scratch_types=[
            pltpu.VMEM((IPW,), jnp.int32),
            pltpu.VMEM((NBUF, GB, WPR), jnp.int32),
            pltpu.VMEM((NBUF, GB, D), jnp.bfloat16),
            (
                pltpu.SemaphoreType.DMA,
                [pltpu.SemaphoreType.DMA] * NBUF,
                [pltpu.SemaphoreType.DMA] * NBUF,
            ),
        ],
        compiler_params=pltpu.CompilerParams(
            needs_layout_passes=False, use_tc_tiling_on_sc=False
        ),
    )
    return kfn(idx_flat, wtab2)


@jax.jit
def _run(input, embedding_weight):
    tabT = jnp.transpose(embedding_weight)
    wtab = _pack_table(tabT)
    wtab2 = wtab.reshape(VS, WPR)
    idx_flat = jnp.transpose(input).reshape(B * H)
    return _gather(idx_flat, wtab2)


def kernel(input, embedding_weight):
    return _run(input, embedding_weight)
